# baseline (speedup 1.0000x reference)
import functools
import math

import jax
import jax.numpy as jnp
import numpy as np
from jax.experimental import pallas as pl
from jax.experimental.pallas import tpu as pltpu
from jax.sharding import Mesh, PartitionSpec as P

try:
    from jax.experimental.shard_map import shard_map as _shard_map
except ImportError:
    _shard_map = jax.shard_map

BB = 256
HB = 64
NCH = 4


def _tanh_sig(x):
    return 0.5 * jnp.tanh(0.5 * x) + 0.5


def _lstm3_kernel(x_ref,
                  wih0_ref, whh0_ref, b0_ref,
                  wih1_ref, whh1_ref, b1_ref,
                  wih2_ref, whh2_ref, b2_ref,
                  wlin_ref, blin_ref,
                  out_ref,
                  seq_a, seq_b,
                  *, T, H):
    f32 = jnp.float32

    bf16 = jnp.bfloat16

    def run_layer(in_ref, out_seq, wih_ref, whh_ref, b_ref):
        def step(t, carry):
            hc = list(carry[:NCH])
            cc = list(carry[NCH:])
            for j in range(2):
                for mxu in range(2):
                    n = 2 * mxu + j
                    pltpu.matmul_push_rhs(wih_ref[:, n * H:(n + 1) * H],
                                          0, mxu)
                for c in range(NCH):
                    row = pl.multiple_of(t * BB + c * HB, HB)
                    in_t = in_ref[pl.ds(row, HB), :]
                    for mxu in range(2):
                        pltpu.matmul_acc_lhs(c * 32 + j * 16, in_t, mxu,
                                             load_staged_rhs=0 if c == 0
                                             else None)
                for mxu in range(2):
                    n = 2 * mxu + j
                    pltpu.matmul_push_rhs(whh_ref[:, n * H:(n + 1) * H],
                                          1, mxu)
                for c in range(NCH):
                    for mxu in range(2):
                        pltpu.matmul_acc_lhs(c * 32 + j * 16, hc[c], mxu,
                                             load_staged_rhs=1 if c == 0
                                             else None)
            for c in range(NCH):
                row = pl.multiple_of(t * BB + c * HB, HB)
                g_i = (pltpu.matmul_pop(c * 32, (HB, H), f32, 0)
                       + b_ref[:, :H])
                g_f = (pltpu.matmul_pop(c * 32 + 16, (HB, H), f32, 0)
                       + b_ref[:, H:2 * H])
                g_g = (pltpu.matmul_pop(c * 32, (HB, H), f32, 1)
                       + b_ref[:, 2 * H:3 * H])
                g_o = (pltpu.matmul_pop(c * 32 + 16, (HB, H), f32, 1)
                       + b_ref[:, 3 * H:])
                c_new = _tanh_sig(g_f) * cc[c] + _tanh_sig(g_i) * jnp.tanh(g_g)
                h_new = _tanh_sig(g_o) * jnp.tanh(c_new)
                hc[c] = h_new.astype(bf16)
                cc[c] = c_new
                if out_seq is not None:
                    out_seq[pl.ds(row, HB), :] = hc[c]
            return tuple(hc) + tuple(cc)

        zh = jnp.zeros((HB, H), bf16)
        zc = jnp.zeros((HB, H), f32)
        carry = jax.lax.fori_loop(0, T, step,
                                  tuple([zh] * NCH) + tuple([zc] * NCH),
                                  unroll=4)
        return list(carry[:NCH])

    I = x_ref.shape[1]
    seq_b[...] = jnp.pad(x_ref[...], ((0, 0), (0, H - I))).astype(bf16)
    run_layer(seq_b, seq_a, wih0_ref, whh0_ref, b0_ref)
    run_layer(seq_a, seq_b, wih1_ref, whh1_ref, b1_ref)
    h_fin = run_layer(seq_b, None, wih2_ref, whh2_ref, b2_ref)

    wlin = wlin_ref[...]
    for c in range(NCH):
        out_ref[pl.ds(c * HB, HB), :] = (
            jnp.sum(h_fin[c].astype(f32) * wlin, axis=-1, keepdims=True)
            + blin_ref[...])


def _forward(x_blocks, lstm_params, lin_w_row, lin_b, *, T, B_pad, H, I):
    nb = B_pad // BB
    body = functools.partial(_lstm3_kernel, T=T, H=H)

    def full2d(shape):
        return pl.BlockSpec(shape, lambda i: (0,) * len(shape))

    in_specs = [pl.BlockSpec((None, T * BB, I), lambda i: (i, 0, 0))]
    args = [x_blocks]
    for (w_ih, w_hh, bias) in lstm_params:
        in_specs += [full2d(w_ih.shape), full2d(w_hh.shape),
                     full2d(bias.shape)]
        args += [w_ih, w_hh, bias]
    in_specs += [full2d((1, H)), full2d((1, 1))]
    args += [lin_w_row, lin_b]

    return pl.pallas_call(
        body,
        out_shape=jax.ShapeDtypeStruct((B_pad, 1), jnp.float32),
        grid=(nb,),
        in_specs=in_specs,
        out_specs=pl.BlockSpec((BB, 1), lambda i: (i, 0)),
        scratch_shapes=[
            pltpu.VMEM((T * BB, H), jnp.bfloat16),
            pltpu.VMEM((T * BB, H), jnp.bfloat16),
        ],
        compiler_params=pltpu.CompilerParams(
            dimension_semantics=("parallel",),
            vmem_limit_bytes=56 * 1024 * 1024),
    )(*args)


@jax.jit
def kernel(x, lstm0_w_ih_t, lstm0_w_hh_t, lstm0_bias,
           lstm1_w_ih_t, lstm1_w_hh_t, lstm1_bias,
           lstm2_w_ih_t, lstm2_w_hh_t, lstm2_bias,
           lin_w_row, lin_b):
    B, T, I = x.shape
    H = lstm0_w_hh_t.shape[0]

    bf16 = jnp.bfloat16
    wih0 = jnp.zeros((H, 4 * H), jnp.float32).at[:I, :].set(lstm0_w_ih_t)
    lstm_params = [
        (wih0.astype(bf16), lstm0_w_hh_t.astype(bf16), lstm0_bias),
        (lstm1_w_ih_t.astype(bf16), lstm1_w_hh_t.astype(bf16), lstm1_bias),
        (lstm2_w_ih_t.astype(bf16), lstm2_w_hh_t.astype(bf16), lstm2_bias),
    ]

    flat_w = [w for lp in lstm_params for w in lp] + [lin_w_row, lin_b]

    def fwd(xs, *ws):
        Bl = xs.shape[0]
        B_pad = ((Bl + BB - 1) // BB) * BB
        nb = B_pad // BB
        x_tm = jnp.transpose(xs.astype(jnp.float32), (1, 0, 2))
        x_tm = jnp.pad(x_tm, ((0, 0), (0, B_pad - Bl), (0, 0)))
        x_blocks = (x_tm.reshape(T, nb, BB, I)
                    .transpose(1, 0, 2, 3)
                    .reshape(nb, T * BB, I))
        lps = [tuple(ws[3 * l:3 * l + 3]) for l in range(3)]
        return _forward(x_blocks, lps, ws[9], ws[10],
                        T=T, B_pad=B_pad, H=H, I=I)

    devs = jax.devices()
    if len(devs) >= 2 and B % (2 * BB) == 0:
        mesh = Mesh(np.array(devs[:2]), ("d",))
        try:
            fwd = _shard_map(
                fwd, mesh=mesh,
                in_specs=(P("d"),) + (P(),) * len(flat_w),
                out_specs=P("d"), check_vma=False)
        except TypeError:
            fwd = _shard_map(
                fwd, mesh=mesh,
                in_specs=(P("d"),) + (P(),) * len(flat_w),
                out_specs=P("d"), check_rep=False)
    out = fwd(x, *flat_w)
    return out[:B, 0]


# speedup vs baseline: 1.1398x; 1.1398x over previous
import functools
import math

import jax
import jax.numpy as jnp
import numpy as np
from jax.experimental import pallas as pl
from jax.experimental.pallas import tpu as pltpu
from jax.sharding import Mesh, PartitionSpec as P

try:
    from jax.experimental.shard_map import shard_map as _shard_map
except ImportError:
    _shard_map = jax.shard_map

BB = 512
HB = 64
NCH = 8


def _tanh_sig(x):
    return 0.5 * jnp.tanh(0.5 * x) + 0.5


def _lstm3_kernel(x_ref,
                  wih0_ref, whh0_ref, b0_ref,
                  wih1_ref, whh1_ref, b1_ref,
                  wih2_ref, whh2_ref, b2_ref,
                  wlin_ref, blin_ref,
                  out_ref,
                  seq_a, seq_b,
                  *, T, H, I):
    f32 = jnp.float32

    bf16 = jnp.bfloat16

    def run_layer(in_ref, out_seq, wih_ref, whh_ref, b_ref, in_packed_i=None):
        if in_packed_i is not None:
            ii = in_packed_i
            lane = jax.lax.broadcasted_iota(jnp.int32, (HB, 16 * ii), 1)
            rk = (jax.lax.broadcasted_iota(jnp.int32, (HB, 16 * ii), 0)
                  % 16) * ii
            xmask = ((lane >= rk) & (lane < rk + ii)).astype(f32)

        def step(t, carry):
            hc = list(carry[:NCH])
            cc = list(carry[NCH:])
            for j in range(2):
                for mxu in range(2):
                    n = 2 * mxu + j
                    pltpu.matmul_push_rhs(wih_ref[:, n * H:(n + 1) * H],
                                          0, mxu)
                for c in range(NCH):
                    row = pl.multiple_of(t * BB + c * HB, HB)
                    if in_packed_i is not None:
                        prow = pl.multiple_of((t * BB + c * HB) // 16,
                                              HB // 16)
                        xp = in_ref[pl.ds(prow, HB // 16), :]
                        xb = jnp.broadcast_to(
                            xp[:, None, :],
                            (HB // 16, 16, 16 * in_packed_i)
                        ).reshape(HB, 16 * in_packed_i)
                        in_t = jnp.pad(
                            xb * xmask,
                            ((0, 0), (0, H - 16 * in_packed_i))
                        ).astype(bf16)
                    else:
                        in_t = in_ref[pl.ds(row, HB), :]
                    for mxu in range(2):
                        pltpu.matmul_acc_lhs(c * 32 + j * 16, in_t, mxu,
                                             load_staged_rhs=0 if c == 0
                                             else None)
                for mxu in range(2):
                    n = 2 * mxu + j
                    pltpu.matmul_push_rhs(whh_ref[:, n * H:(n + 1) * H],
                                          1, mxu)
                for c in range(NCH):
                    for mxu in range(2):
                        pltpu.matmul_acc_lhs(c * 32 + j * 16, hc[c], mxu,
                                             load_staged_rhs=1 if c == 0
                                             else None)
            for c in range(NCH):
                row = pl.multiple_of(t * BB + c * HB, HB)
                g_i = (pltpu.matmul_pop(c * 32, (HB, H), f32, 0)
                       + b_ref[:, :H])
                g_f = (pltpu.matmul_pop(c * 32 + 16, (HB, H), f32, 0)
                       + b_ref[:, H:2 * H])
                g_g = (pltpu.matmul_pop(c * 32, (HB, H), f32, 1)
                       + b_ref[:, 2 * H:3 * H])
                g_o = (pltpu.matmul_pop(c * 32 + 16, (HB, H), f32, 1)
                       + b_ref[:, 3 * H:])
                c_new = _tanh_sig(g_f) * cc[c] + _tanh_sig(g_i) * jnp.tanh(g_g)
                h_new = _tanh_sig(g_o) * jnp.tanh(c_new)
                hc[c] = h_new.astype(bf16)
                cc[c] = c_new
                if out_seq is not None:
                    out_seq[pl.ds(row, HB), :] = hc[c]
            return tuple(hc) + tuple(cc)

        zh = jnp.zeros((HB, H), bf16)
        zc = jnp.zeros((HB, H), f32)
        carry = jax.lax.fori_loop(0, T, step,
                                  tuple([zh] * NCH) + tuple([zc] * NCH),
                                  unroll=4)
        return list(carry[:NCH])

    run_layer(x_ref, seq_a, wih0_ref, whh0_ref, b0_ref, in_packed_i=I)
    run_layer(seq_a, seq_b, wih1_ref, whh1_ref, b1_ref)
    h_fin = run_layer(seq_b, None, wih2_ref, whh2_ref, b2_ref)

    wlin = wlin_ref[...]
    for c in range(NCH):
        out_ref[pl.ds(c * HB, HB), :] = (
            jnp.sum(h_fin[c].astype(f32) * wlin, axis=-1, keepdims=True)
            + blin_ref[...])


def _forward(x_blocks, lstm_params, lin_w_row, lin_b, *, T, B_pad, H, I):
    nb = B_pad // BB
    body = functools.partial(_lstm3_kernel, T=T, H=H, I=I)

    def full2d(shape):
        return pl.BlockSpec(shape, lambda i: (0,) * len(shape))

    in_specs = [pl.BlockSpec((None, T * BB // 16, 16 * I),
                             lambda i: (i, 0, 0))]
    args = [x_blocks]
    for (w_ih, w_hh, bias) in lstm_params:
        in_specs += [full2d(w_ih.shape), full2d(w_hh.shape),
                     full2d(bias.shape)]
        args += [w_ih, w_hh, bias]
    in_specs += [full2d((1, H)), full2d((1, 1))]
    args += [lin_w_row, lin_b]

    return pl.pallas_call(
        body,
        out_shape=jax.ShapeDtypeStruct((B_pad, 1), jnp.float32),
        grid=(nb,),
        in_specs=in_specs,
        out_specs=pl.BlockSpec((BB, 1), lambda i: (i, 0)),
        scratch_shapes=[
            pltpu.VMEM((T * BB, H), jnp.bfloat16),
            pltpu.VMEM((T * BB, H), jnp.bfloat16),
        ],
        compiler_params=pltpu.CompilerParams(
            dimension_semantics=("parallel",),
            vmem_limit_bytes=56 * 1024 * 1024),
    )(*args)


@jax.jit
def kernel(x, lstm0_w_ih_t, lstm0_w_hh_t, lstm0_bias,
           lstm1_w_ih_t, lstm1_w_hh_t, lstm1_bias,
           lstm2_w_ih_t, lstm2_w_hh_t, lstm2_bias,
           lin_w_row, lin_b):
    B, T, I = x.shape
    H = lstm0_w_hh_t.shape[0]

    bf16 = jnp.bfloat16
    wih0 = jnp.zeros((H, 4 * H), jnp.float32).at[:16 * I, :].set(
        jnp.tile(lstm0_w_ih_t, (16, 1)))
    lstm_params = [
        (wih0.astype(bf16), lstm0_w_hh_t.astype(bf16), lstm0_bias),
        (lstm1_w_ih_t.astype(bf16), lstm1_w_hh_t.astype(bf16), lstm1_bias),
        (lstm2_w_ih_t.astype(bf16), lstm2_w_hh_t.astype(bf16), lstm2_bias),
    ]

    flat_w = [w for lp in lstm_params for w in lp] + [lin_w_row, lin_b]

    def fwd(xs, *ws):
        Bl = xs.shape[0]
        B_pad = ((Bl + BB - 1) // BB) * BB
        nb = B_pad // BB
        x_tm = jnp.transpose(xs.astype(jnp.float32), (1, 0, 2))
        x_tm = jnp.pad(x_tm, ((0, 0), (0, B_pad - Bl), (0, 0)))
        x_blocks = (x_tm.reshape(T, nb, BB, I)
                    .transpose(1, 0, 2, 3)
                    .reshape(nb, T * BB // 16, 16 * I))
        lps = [tuple(ws[3 * l:3 * l + 3]) for l in range(3)]
        return _forward(x_blocks, lps, ws[9], ws[10],
                        T=T, B_pad=B_pad, H=H, I=I)

    devs = jax.devices()
    if len(devs) >= 2 and B % (2 * BB) == 0:
        mesh = Mesh(np.array(devs[:2]), ("d",))
        try:
            fwd = _shard_map(
                fwd, mesh=mesh,
                in_specs=(P("d"),) + (P(),) * len(flat_w),
                out_specs=P("d"), check_vma=False)
        except TypeError:
            fwd = _shard_map(
                fwd, mesh=mesh,
                in_specs=(P("d"),) + (P(),) * len(flat_w),
                out_specs=P("d"), check_rep=False)
    out = fwd(x, *flat_w)
    return out[:B, 0]


# speedup vs baseline: 1.1580x; 1.0159x over previous
import functools
import math

import jax
import jax.numpy as jnp
import numpy as np
from jax.experimental import pallas as pl
from jax.experimental.pallas import tpu as pltpu
from jax.sharding import Mesh, PartitionSpec as P

try:
    from jax.experimental.shard_map import shard_map as _shard_map
except ImportError:
    _shard_map = jax.shard_map

BB = 512
HB = 64
NCH = 8


def _tanh_sig(x):
    return 0.5 * jnp.tanh(0.5 * x) + 0.5


def _lstm3_kernel(x_ref,
                  wih0_ref, whh0_ref, b0_ref,
                  wih1_ref, whh1_ref, b1_ref,
                  wih2_ref, whh2_ref, b2_ref,
                  wlin_ref, blin_ref,
                  out_ref,
                  seq_a, seq_b,
                  *, T, H, I):
    f32 = jnp.float32

    bf16 = jnp.bfloat16

    def run_layer(in_ref, out_seq, wih_ref, whh_ref, b_ref, in_packed_i=None):
        if in_packed_i is not None:
            ii = in_packed_i
            lane = jax.lax.broadcasted_iota(jnp.int32, (HB, 16 * ii), 1)
            rk = (jax.lax.broadcasted_iota(jnp.int32, (HB, 16 * ii), 0)
                  % 16) * ii
            xmask = ((lane >= rk) & (lane < rk + ii)).astype(f32)

        def step(t, carry):
            hc = list(carry[:NCH])
            cc = list(carry[NCH:])
            hcb = [h.astype(bf16) for h in hc]
            for j in range(2):
                for mxu in range(2):
                    n = 2 * mxu + j
                    pltpu.matmul_push_rhs(wih_ref[:, n * H:(n + 1) * H],
                                          0, mxu)
                for c in range(NCH):
                    row = pl.multiple_of(t * BB + c * HB, HB)
                    if in_packed_i is not None:
                        prow = pl.multiple_of((t * BB + c * HB) // 16,
                                              HB // 16)
                        xp = in_ref[pl.ds(prow, HB // 16), :]
                        xb = jnp.broadcast_to(
                            xp[:, None, :],
                            (HB // 16, 16, 16 * in_packed_i)
                        ).reshape(HB, 16 * in_packed_i)
                        in_t = jnp.pad(
                            xb * xmask,
                            ((0, 0), (0, H - 16 * in_packed_i))
                        ).astype(bf16)
                    else:
                        in_t = in_ref[pl.ds(row, HB), :]
                    for mxu in range(2):
                        pltpu.matmul_acc_lhs(c * 32 + j * 16, in_t, mxu,
                                             load_staged_rhs=0 if c == 0
                                             else None)
                for mxu in range(2):
                    n = 2 * mxu + j
                    pltpu.matmul_push_rhs(whh_ref[:, n * H:(n + 1) * H],
                                          1, mxu)
                for c in range(NCH):
                    for mxu in range(2):
                        pltpu.matmul_acc_lhs(c * 32 + j * 16, hcb[c], mxu,
                                             load_staged_rhs=1 if c == 0
                                             else None)
            for c in range(NCH):
                row = pl.multiple_of(t * BB + c * HB, HB)
                g_i = (pltpu.matmul_pop(c * 32, (HB, H), f32, 0)
                       + b_ref[:, :H])
                g_f = (pltpu.matmul_pop(c * 32 + 16, (HB, H), f32, 0)
                       + b_ref[:, H:2 * H])
                g_g = (pltpu.matmul_pop(c * 32, (HB, H), f32, 1)
                       + b_ref[:, 2 * H:3 * H])
                g_o = (pltpu.matmul_pop(c * 32 + 16, (HB, H), f32, 1)
                       + b_ref[:, 3 * H:])
                c_new = _tanh_sig(g_f) * cc[c] + _tanh_sig(g_i) * jnp.tanh(g_g)
                h_new = _tanh_sig(g_o) * jnp.tanh(c_new)
                hc[c] = h_new
                cc[c] = c_new
                if out_seq is not None:
                    out_seq[pl.ds(row, HB), :] = h_new.astype(bf16)
            return tuple(hc) + tuple(cc)

        zh = jnp.zeros((HB, H), f32)
        zc = jnp.zeros((HB, H), f32)
        carry = jax.lax.fori_loop(0, T, step,
                                  tuple([zh] * NCH) + tuple([zc] * NCH),
                                  unroll=4)
        return list(carry[:NCH])

    run_layer(x_ref, seq_a, wih0_ref, whh0_ref, b0_ref, in_packed_i=I)
    run_layer(seq_a, seq_b, wih1_ref, whh1_ref, b1_ref)
    h_fin = run_layer(seq_b, None, wih2_ref, whh2_ref, b2_ref)

    wlin = wlin_ref[...]
    for c in range(NCH):
        out_ref[pl.ds(c * HB, HB), :] = (
            jnp.sum(h_fin[c] * wlin, axis=-1, keepdims=True)
            + blin_ref[...])


def _forward(x_blocks, lstm_params, lin_w_row, lin_b, *, T, B_pad, H, I):
    nb = B_pad // BB
    body = functools.partial(_lstm3_kernel, T=T, H=H, I=I)

    def full2d(shape):
        return pl.BlockSpec(shape, lambda i: (0,) * len(shape))

    in_specs = [pl.BlockSpec((None, T * BB // 16, 16 * I),
                             lambda i: (i, 0, 0))]
    args = [x_blocks]
    for (w_ih, w_hh, bias) in lstm_params:
        in_specs += [full2d(w_ih.shape), full2d(w_hh.shape),
                     full2d(bias.shape)]
        args += [w_ih, w_hh, bias]
    in_specs += [full2d((1, H)), full2d((1, 1))]
    args += [lin_w_row, lin_b]

    return pl.pallas_call(
        body,
        out_shape=jax.ShapeDtypeStruct((B_pad, 1), jnp.float32),
        grid=(nb,),
        in_specs=in_specs,
        out_specs=pl.BlockSpec((BB, 1), lambda i: (i, 0)),
        scratch_shapes=[
            pltpu.VMEM((T * BB, H), jnp.bfloat16),
            pltpu.VMEM((T * BB, H), jnp.bfloat16),
        ],
        compiler_params=pltpu.CompilerParams(
            dimension_semantics=("parallel",),
            vmem_limit_bytes=56 * 1024 * 1024),
    )(*args)


@jax.jit
def kernel(x, lstm0_w_ih_t, lstm0_w_hh_t, lstm0_bias,
           lstm1_w_ih_t, lstm1_w_hh_t, lstm1_bias,
           lstm2_w_ih_t, lstm2_w_hh_t, lstm2_bias,
           lin_w_row, lin_b):
    B, T, I = x.shape
    H = lstm0_w_hh_t.shape[0]

    bf16 = jnp.bfloat16
    wih0 = jnp.zeros((H, 4 * H), jnp.float32).at[:16 * I, :].set(
        jnp.tile(lstm0_w_ih_t, (16, 1)))
    lstm_params = [
        (wih0.astype(bf16), lstm0_w_hh_t.astype(bf16), lstm0_bias),
        (lstm1_w_ih_t.astype(bf16), lstm1_w_hh_t.astype(bf16), lstm1_bias),
        (lstm2_w_ih_t.astype(bf16), lstm2_w_hh_t.astype(bf16), lstm2_bias),
    ]

    flat_w = [w for lp in lstm_params for w in lp] + [lin_w_row, lin_b]

    def fwd(xs, *ws):
        Bl = xs.shape[0]
        B_pad = ((Bl + BB - 1) // BB) * BB
        nb = B_pad // BB
        x_tm = jnp.transpose(xs.astype(jnp.float32), (1, 0, 2))
        x_tm = jnp.pad(x_tm, ((0, 0), (0, B_pad - Bl), (0, 0)))
        x_blocks = (x_tm.reshape(T, nb, BB, I)
                    .transpose(1, 0, 2, 3)
                    .reshape(nb, T * BB // 16, 16 * I))
        lps = [tuple(ws[3 * l:3 * l + 3]) for l in range(3)]
        return _forward(x_blocks, lps, ws[9], ws[10],
                        T=T, B_pad=B_pad, H=H, I=I)

    devs = jax.devices()
    if len(devs) >= 2 and B % (2 * BB) == 0:
        mesh = Mesh(np.array(devs[:2]), ("d",))
        try:
            fwd = _shard_map(
                fwd, mesh=mesh,
                in_specs=(P("d"),) + (P(),) * len(flat_w),
                out_specs=P("d"), check_vma=False)
        except TypeError:
            fwd = _shard_map(
                fwd, mesh=mesh,
                in_specs=(P("d"),) + (P(),) * len(flat_w),
                out_specs=P("d"), check_rep=False)
    out = fwd(x, *flat_w)
    return out[:B, 0]


# speedup vs baseline: 1.2081x; 1.0432x over previous
import functools
import math

import jax
import jax.numpy as jnp
import numpy as np
from jax.experimental import pallas as pl
from jax.experimental.pallas import tpu as pltpu
from jax.sharding import Mesh, PartitionSpec as P

try:
    from jax.experimental.shard_map import shard_map as _shard_map
except ImportError:
    _shard_map = jax.shard_map

BB = 512
HB = 64
NCH = 8


def _tanh_sig(x):
    return 0.5 * jnp.tanh(0.5 * x) + 0.5


def _lstm3_kernel(x_ref,
                  wih0_ref, whh0_ref, b0_ref,
                  wih1_ref, whh1_ref, b1_ref,
                  wih2_ref, whh2_ref, b2_ref,
                  wlin_ref, blin_ref,
                  out_ref,
                  seq_a, seq_b,
                  *, T, H, I):
    f32 = jnp.float32

    bf16 = jnp.bfloat16

    def run_layer(in_ref, out_seq, wih_ref, whh_ref, b_ref, in_packed_i=None):
        if in_packed_i is not None:
            ii = in_packed_i
            lane = jax.lax.broadcasted_iota(jnp.int32, (HB, 16 * ii), 1)
            rk = (jax.lax.broadcasted_iota(jnp.int32, (HB, 16 * ii), 0)
                  % 16) * ii
            xmask = ((lane >= rk) & (lane < rk + ii)).astype(f32)

        def step(t, carry):
            hc = list(carry[:NCH])
            cc = list(carry[NCH:])
            hcb = [h.astype(bf16) for h in hc]
            for j in range(2):
                for mxu in range(2):
                    n = 2 * mxu + j
                    pltpu.matmul_push_rhs(wih_ref[:, n * H:(n + 1) * H],
                                          0, mxu)
                for c in range(NCH):
                    row = pl.multiple_of(t * BB + c * HB, HB)
                    if in_packed_i is not None:
                        prow = pl.multiple_of((t * BB + c * HB) // 16,
                                              HB // 16)
                        xp = in_ref[pl.ds(prow, HB // 16), :]
                        xb = jnp.broadcast_to(
                            xp[:, None, :],
                            (HB // 16, 16, 16 * in_packed_i)
                        ).reshape(HB, 16 * in_packed_i)
                        in_t = jnp.pad(
                            xb * xmask,
                            ((0, 0), (0, H - 16 * in_packed_i))
                        ).astype(bf16)
                    else:
                        in_t = in_ref[pl.ds(row, HB), :]
                    for mxu in range(2):
                        pltpu.matmul_acc_lhs(c * 32 + j * 16, in_t, mxu,
                                             load_staged_rhs=0 if c == 0
                                             else None)
                for mxu in range(2):
                    n = 2 * mxu + j
                    pltpu.matmul_push_rhs(whh_ref[:, n * H:(n + 1) * H],
                                          1, mxu)
                for c in range(NCH):
                    for mxu in range(2):
                        pltpu.matmul_acc_lhs(c * 32 + j * 16, hcb[c], mxu,
                                             load_staged_rhs=1 if c == 0
                                             else None)
            for c in range(NCH):
                row = pl.multiple_of(t * BB + c * HB, HB)
                g_i = (pltpu.matmul_pop(c * 32, (HB, H), f32, 0)
                       + b_ref[:, :H])
                g_f = (pltpu.matmul_pop(c * 32 + 16, (HB, H), f32, 0)
                       + b_ref[:, H:2 * H])
                g_g = (pltpu.matmul_pop(c * 32, (HB, H), f32, 1)
                       + b_ref[:, 2 * H:3 * H])
                g_o = (pltpu.matmul_pop(c * 32 + 16, (HB, H), f32, 1)
                       + b_ref[:, 3 * H:])
                c_new = _tanh_sig(g_f) * cc[c] + _tanh_sig(g_i) * jnp.tanh(g_g)
                h_new = _tanh_sig(g_o) * jnp.tanh(c_new)
                hc[c] = h_new
                cc[c] = c_new
                if out_seq is not None:
                    out_seq[pl.ds(row, HB), :] = h_new.astype(bf16)
            return tuple(hc) + tuple(cc)

        zh = jnp.zeros((HB, H), f32)
        zc = jnp.zeros((HB, H), f32)
        carry = jax.lax.fori_loop(0, T, step,
                                  tuple([zh] * NCH) + tuple([zc] * NCH),
                                  unroll=8)
        return list(carry[:NCH])

    run_layer(x_ref, seq_a, wih0_ref, whh0_ref, b0_ref, in_packed_i=I)
    run_layer(seq_a, seq_b, wih1_ref, whh1_ref, b1_ref)
    h_fin = run_layer(seq_b, None, wih2_ref, whh2_ref, b2_ref)

    wlin = wlin_ref[...]
    for c in range(NCH):
        out_ref[pl.ds(c * HB, HB), :] = (
            jnp.sum(h_fin[c] * wlin, axis=-1, keepdims=True)
            + blin_ref[...])


def _forward(x_blocks, lstm_params, lin_w_row, lin_b, *, T, B_pad, H, I):
    nb = B_pad // BB
    body = functools.partial(_lstm3_kernel, T=T, H=H, I=I)

    def full2d(shape):
        return pl.BlockSpec(shape, lambda i: (0,) * len(shape))

    in_specs = [pl.BlockSpec((None, T * BB // 16, 16 * I),
                             lambda i: (i, 0, 0))]
    args = [x_blocks]
    for (w_ih, w_hh, bias) in lstm_params:
        in_specs += [full2d(w_ih.shape), full2d(w_hh.shape),
                     full2d(bias.shape)]
        args += [w_ih, w_hh, bias]
    in_specs += [full2d((1, H)), full2d((1, 1))]
    args += [lin_w_row, lin_b]

    return pl.pallas_call(
        body,
        out_shape=jax.ShapeDtypeStruct((B_pad, 1), jnp.float32),
        grid=(nb,),
        in_specs=in_specs,
        out_specs=pl.BlockSpec((BB, 1), lambda i: (i, 0)),
        scratch_shapes=[
            pltpu.VMEM((T * BB, H), jnp.bfloat16),
            pltpu.VMEM((T * BB, H), jnp.bfloat16),
        ],
        compiler_params=pltpu.CompilerParams(
            dimension_semantics=("parallel",),
            vmem_limit_bytes=56 * 1024 * 1024),
    )(*args)


@jax.jit
def kernel(x, lstm0_w_ih_t, lstm0_w_hh_t, lstm0_bias,
           lstm1_w_ih_t, lstm1_w_hh_t, lstm1_bias,
           lstm2_w_ih_t, lstm2_w_hh_t, lstm2_bias,
           lin_w_row, lin_b):
    B, T, I = x.shape
    H = lstm0_w_hh_t.shape[0]

    bf16 = jnp.bfloat16
    wih0 = jnp.zeros((H, 4 * H), jnp.float32).at[:16 * I, :].set(
        jnp.tile(lstm0_w_ih_t, (16, 1)))
    lstm_params = [
        (wih0.astype(bf16), lstm0_w_hh_t.astype(bf16), lstm0_bias),
        (lstm1_w_ih_t.astype(bf16), lstm1_w_hh_t.astype(bf16), lstm1_bias),
        (lstm2_w_ih_t.astype(bf16), lstm2_w_hh_t.astype(bf16), lstm2_bias),
    ]

    flat_w = [w for lp in lstm_params for w in lp] + [lin_w_row, lin_b]

    def fwd(xs, *ws):
        Bl = xs.shape[0]
        B_pad = ((Bl + BB - 1) // BB) * BB
        nb = B_pad // BB
        x_tm = jnp.transpose(xs.astype(jnp.float32), (1, 0, 2))
        x_tm = jnp.pad(x_tm, ((0, 0), (0, B_pad - Bl), (0, 0)))
        x_blocks = (x_tm.reshape(T, nb, BB, I)
                    .transpose(1, 0, 2, 3)
                    .reshape(nb, T * BB // 16, 16 * I))
        lps = [tuple(ws[3 * l:3 * l + 3]) for l in range(3)]
        return _forward(x_blocks, lps, ws[9], ws[10],
                        T=T, B_pad=B_pad, H=H, I=I)

    devs = jax.devices()
    if len(devs) >= 2 and B % (2 * BB) == 0:
        mesh = Mesh(np.array(devs[:2]), ("d",))
        try:
            fwd = _shard_map(
                fwd, mesh=mesh,
                in_specs=(P("d"),) + (P(),) * len(flat_w),
                out_specs=P("d"), check_vma=False)
        except TypeError:
            fwd = _shard_map(
                fwd, mesh=mesh,
                in_specs=(P("d"),) + (P(),) * len(flat_w),
                out_specs=P("d"), check_rep=False)
    out = fwd(x, *flat_w)
    return out[:B, 0]
